# baseline (speedup 1.0000x reference)
import jax
import jax.numpy as jnp
from jax import lax
from jax.experimental import pallas as pl
from jax.experimental.pallas import tpu as pltpu
from jax.experimental.pallas import tpu_sc as plsc

NUM_MODELS = 2
NUM_BOXES = 100000
DIM = 64
BATCH = 16384

_L = 16
_NC, _NS = 2, 16
_NW = _NC * _NS
_BPW = BATCH // _NW
_CH = 128
_NPH = (_BPW // _CH) * NUM_MODELS
_WR = NUM_BOXES // 2


def _sc_body(zw, ldw, ids, out, ids_v, zb, ldb, mnb, mxb, gsems, wsem):
    wid = lax.axis_index("s") * _NC + lax.axis_index("c")
    base = wid * _BPW
    pltpu.sync_copy(ids.at[pl.ds(base, _BPW)], ids_v)
    hsrc = zw.at[pl.ds(0, _CH)]
    hsr2 = out.at[0, 0, pl.ds(0, _CH)]

    def fire(k):
        ch, m, p = k // 2, k % 2, k % 2

        def go(g, _):
            v = ids_v[pl.ds(ch * _CH + g * _L, _L)]
            w = (v >> 1) + m * _WR
            for j in range(_L):
                dst = pl.ds(g * _L + j, 1)
                row = pl.ds(w[j], 1)
                pltpu.async_copy(zw.at[row], zb.at[p, dst], gsems.at[p])
                pltpu.async_copy(ldw.at[row], ldb.at[p, dst], gsems.at[p])
            return 0

        lax.fori_loop(0, _CH // _L, go, 0)

    fire(0)
    for k in range(_NPH):
        ch, m, p = k // 2, k % 2, k % 2
        if k + 1 < _NPH:
            fire(k + 1)

        pltpu.make_async_copy(hsrc, zb.at[p], gsems.at[p]).wait()
        pltpu.make_async_copy(hsrc, ldb.at[p], gsems.at[p]).wait()
        if k >= 1:
            pltpu.make_async_copy(hsr2, mnb, wsem).wait()
            pltpu.make_async_copy(hsr2, mxb, wsem).wait()

        def body(g, _):
            v = ids_v[pl.ds(ch * _CH + g * _L, _L)]
            par = v & 1
            for j in range(_L):
                odd = par[j] == 1
                r = g * _L + j
                for c in range(DIM // _L):
                    lo = pl.ds(c * _L, _L)
                    hi = pl.ds(DIM + c * _L, _L)
                    zv = jnp.where(odd, zb[p, r, hi], zb[p, r, lo])
                    lv = jnp.where(odd, ldb[p, r, hi], ldb[p, r, lo])
                    mnb[r, lo] = zv
                    mxb[r, lo] = zv + jnp.exp(lv)
            return 0

        lax.fori_loop(0, _CH // _L, body, 0)

        orow = pl.ds(base + ch * _CH, _CH)
        pltpu.async_copy(mnb, out.at[0, m, orow], wsem)
        pltpu.async_copy(mxb, out.at[1, m, orow], wsem)

    pltpu.make_async_copy(hsr2, mnb, wsem).wait()
    pltpu.make_async_copy(hsr2, mxb, wsem).wait()


def kernel(z, logdelta, ids):
    zw = z.reshape(_WR * NUM_MODELS, 2 * DIM)
    ldw = logdelta.reshape(_WR * NUM_MODELS, 2 * DIM)
    ids32 = ids.astype(jnp.int32)
    mesh = plsc.VectorSubcoreMesh(core_axis_name="c", subcore_axis_name="s")
    out = pl.kernel(
        _sc_body,
        mesh=mesh,
        out_type=jax.ShapeDtypeStruct((2, NUM_MODELS, BATCH, DIM), jnp.float32),
        scratch_types=[
            pltpu.VMEM((_BPW,), jnp.int32),
            pltpu.VMEM((2, _CH, 2 * DIM), jnp.float32),
            pltpu.VMEM((2, _CH, 2 * DIM), jnp.float32),
            pltpu.VMEM((_CH, DIM), jnp.float32),
            pltpu.VMEM((_CH, DIM), jnp.float32),
            pltpu.SemaphoreType.DMA((2,)),
            pltpu.SemaphoreType.DMA,
        ],
    )(zw, ldw, ids32)
    return out


# speedup vs baseline: 1.5828x; 1.5828x over previous
import jax
import jax.numpy as jnp
from jax import lax
from jax.experimental import pallas as pl
from jax.experimental.pallas import tpu as pltpu
from jax.experimental.pallas import tpu_sc as plsc

NUM_MODELS = 2
NUM_BOXES = 100000
DIM = 64
BATCH = 16384

_L = 16
_NC, _NS = 2, 16
_NW = _NC * _NS
_BPW = BATCH // _NW
_CH = 64
_NCH = _BPW // _CH


def _sc_body(z, ld, ids, out, ids_v, zb, ldb, gsems, wsems):
    wid = lax.axis_index("s") * _NC + lax.axis_index("c")
    base = wid * _BPW
    pltpu.sync_copy(ids.at[pl.ds(base, _BPW)], ids_v)
    hsrc = z.at[0].at[pl.ds(0, _CH)]

    def fire(ch):
        p = ch % 2

        def go(g, _):
            v = ids_v[pl.ds(ch * _CH + g * _L, _L)]
            for j in range(_L):
                row = pl.ds(v[j], 1)
                dst = pl.ds(g * _L + j, 1)
                for m in range(NUM_MODELS):
                    pltpu.async_copy(z.at[m].at[row], zb.at[p, m, dst], gsems.at[p])
                    pltpu.async_copy(ld.at[m].at[row], ldb.at[p, m, dst], gsems.at[p])
            return 0

        lax.fori_loop(0, _CH // _L, go, 0)

    fire(0)
    for ch in range(_NCH):
        p = ch % 2
        if ch + 1 < _NCH:
            p2 = (ch + 1) % 2
            if ch + 1 >= 2:
                for m in range(NUM_MODELS):
                    pltpu.make_async_copy(hsrc, zb.at[p2, m], wsems.at[p2]).wait()
                    pltpu.make_async_copy(hsrc, ldb.at[p2, m], wsems.at[p2]).wait()
            fire(ch + 1)

        for m in range(NUM_MODELS):
            pltpu.make_async_copy(hsrc, zb.at[p, m], gsems.at[p]).wait()
            pltpu.make_async_copy(hsrc, ldb.at[p, m], gsems.at[p]).wait()

        def body(r, _):
            for m in range(NUM_MODELS):
                for c in range(DIM // _L):
                    s = pl.ds(c * _L, _L)
                    ldb[p, m, r, s] = zb[p, m, r, s] + jnp.exp(ldb[p, m, r, s])
            return 0

        lax.fori_loop(0, _CH, body, 0)

        orow = pl.ds(base + ch * _CH, _CH)
        for m in range(NUM_MODELS):
            pltpu.async_copy(zb.at[p, m], out.at[0, m, orow], wsems.at[p])
            pltpu.async_copy(ldb.at[p, m], out.at[1, m, orow], wsems.at[p])

    for p in range(2):
        for m in range(NUM_MODELS):
            pltpu.make_async_copy(hsrc, zb.at[p, m], wsems.at[p]).wait()
            pltpu.make_async_copy(hsrc, ldb.at[p, m], wsems.at[p]).wait()


def kernel(z, logdelta, ids):
    ids32 = ids.astype(jnp.int32)
    mesh = plsc.VectorSubcoreMesh(core_axis_name="c", subcore_axis_name="s")
    out = pl.kernel(
        _sc_body,
        mesh=mesh,
        out_type=jax.ShapeDtypeStruct((2, NUM_MODELS, BATCH, DIM), jnp.float32),
        scratch_types=[
            pltpu.VMEM((_BPW,), jnp.int32),
            pltpu.VMEM((2, NUM_MODELS, _CH, DIM), jnp.float32),
            pltpu.VMEM((2, NUM_MODELS, _CH, DIM), jnp.float32),
            pltpu.SemaphoreType.DMA((2,)),
            pltpu.SemaphoreType.DMA((2,)),
        ],
    )(z, logdelta, ids32)
    return out
